# Initial kernel scaffold; baseline (speedup 1.0000x reference)
import jax
import jax.numpy as jnp
from jax.experimental import pallas as pl


def kernel(x, adj, W0, bias):
    raise NotImplementedError("write your pallas kernel here")


# speedup vs baseline: 1.0414x; 1.0414x over previous
import functools

import jax
import jax.numpy as jnp
from jax.experimental import pallas as pl


def _cheb_block(x_blk_ref, adj_ref, x_ref, w_ref, b_ref, o_ref):
    acc = jnp.dot(adj_ref[...], x_ref[...], preferred_element_type=jnp.float32)
    t = x_blk_ref[...] + acc
    o_ref[...] = jnp.dot(t, w_ref[...], preferred_element_type=jnp.float32) + b_ref[...]


@functools.partial(jax.jit, static_argnames=())
def kernel(x, adj, W0, bias):
    n, d_in = x.shape
    d_out = W0.shape[1]
    blk = 400
    assert n % blk == 0
    grid = (n // blk,)
    b2d = bias.reshape(1, d_out)
    return pl.pallas_call(
        _cheb_block,
        grid=grid,
        in_specs=[
            pl.BlockSpec((blk, d_in), lambda i: (i, 0)),
            pl.BlockSpec((blk, n), lambda i: (i, 0)),
            pl.BlockSpec((n, d_in), lambda i: (0, 0)),
            pl.BlockSpec((d_in, d_out), lambda i: (0, 0)),
            pl.BlockSpec((1, d_out), lambda i: (0, 0)),
        ],
        out_specs=pl.BlockSpec((blk, d_out), lambda i: (i, 0)),
        out_shape=jax.ShapeDtypeStruct((n, d_out), x.dtype),
    )(x, adj, x, W0, b2d)
